# baseline (speedup 1.0000x reference)
import dataclasses
import functools

import jax
import jax.numpy as jnp
from jax import lax
from jax.experimental import pallas as pl
from jax.experimental.pallas import tpu as pltpu
from jax.experimental.pallas import tpu_sc as plsc

_N = 10000
_E = 320000
_T = 4
_NP = 10240
_TRASH = 10000
_NW = 32
_B = 128
_NCHUNK = 80
_EPW = _NCHUNK * _B
_EPAD = _NW * _EPW
_EROWS = _EPAD // 128
_SCALE = 1.0 / 128.0


def _prep_body(src_ref, u_ref, v_ref, gidx_ref, coeff_ref):
    inv_sp = (_T - 1) / 2.0
    tu = (u_ref[...] + 1.0) * inv_sp
    p = jnp.clip(jnp.floor(tu), 0.0, _T - 2.0)
    fu = tu - p
    tv = (v_ref[...] + 1.0) * inv_sp
    q = jnp.clip(jnp.floor(tv), 0.0, _T - 2.0)
    fv = tv - q
    base = src_ref[...] * 16 + p.astype(jnp.int32) * 4 + q.astype(jnp.int32)
    gu = 1.0 - fu
    gv = 1.0 - fv
    gidx_ref[0] = base
    gidx_ref[1] = base + 1
    gidx_ref[2] = base + 4
    gidx_ref[3] = base + 5
    coeff_ref[0] = gu * gv
    coeff_ref[1] = gu * fv
    coeff_ref[2] = fu * gv
    coeff_ref[3] = fu * fv


_prep_call = pl.pallas_call(
    _prep_body,
    out_shape=(
        jax.ShapeDtypeStruct((4, _EROWS, 128), jnp.int32),
        jax.ShapeDtypeStruct((4, _EROWS, 128), jnp.float32),
    ),
)


def _pack_words(o):
    half = o.shape[-1] // 2
    lo = o[:, :half].astype(jnp.bfloat16).astype(jnp.float32)
    hi = o[:, half:].astype(jnp.bfloat16).astype(jnp.float32)
    lo_b = lax.shift_right_logical(
        lax.bitcast_convert_type(lo, jnp.int32), 16)
    hi_b = lax.bitcast_convert_type(hi, jnp.int32) & jnp.int32(-65536)
    return lo_b | hi_b


def _mm_first_body(a_ref, w_ref, o_ref):
    o = jnp.dot(a_ref[...], w_ref[...], preferred_element_type=jnp.float32)
    o_ref[...] = _pack_words(o)


def _mm_fused_body(p_ref, w_ref, o_ref, *, scale):
    a = jnp.maximum(p_ref[0] + p_ref[1], 0.0)
    o = jnp.dot(a, w_ref[...], preferred_element_type=jnp.float32)
    if scale != 1.0:
        o = o * scale
    o_ref[...] = _pack_words(o)


_MM_R = 1024


def _make_mm_first(din, fout):
    return pl.pallas_call(
        _mm_first_body,
        grid=(_NP // _MM_R,),
        in_specs=[
            pl.BlockSpec((_MM_R, din), lambda i: (i, 0)),
            pl.BlockSpec((din, fout), lambda i: (0, 0)),
        ],
        out_specs=pl.BlockSpec((_MM_R, fout // 2), lambda i: (i, 0)),
        out_shape=jax.ShapeDtypeStruct((_NP, fout // 2), jnp.int32),
    )


def _make_mm_fused(din, fout, scale):
    return pl.pallas_call(
        functools.partial(_mm_fused_body, scale=scale),
        grid=(_NP // _MM_R,),
        in_specs=[
            pl.BlockSpec((2, _MM_R, din), lambda i: (0, i, 0)),
            pl.BlockSpec((din, fout), lambda i: (0, 0)),
        ],
        out_specs=pl.BlockSpec((_MM_R, fout // 2), lambda i: (i, 0)),
        out_shape=jax.ShapeDtypeStruct((_NP, fout // 2), jnp.int32),
    )


def _final_add_body(p_ref, o_ref):
    o_ref[...] = p_ref[0] + p_ref[1]


_final_add = pl.pallas_call(
    _final_add_body,
    out_shape=jax.ShapeDtypeStruct((_NP, 32), jnp.float32),
)


def _make_edge_call(F, k0, k1):
    nw = F // 2
    ngrp = F // 32
    stripe_rows = _NP // 16
    mesh = plsc.VectorSubcoreMesh(
        core_axis_name="c", subcore_axis_name="s",
        num_cores=2, num_subcores=16)
    cp = pltpu.CompilerParams()
    if "needs_layout_passes" in pltpu.CompilerParams.__dataclass_fields__:
        cp = dataclasses.replace(cp, needs_layout_passes=False)
    if "use_tc_tiling_on_sc" in pltpu.CompilerParams.__dataclass_fields__:
        cp = dataclasses.replace(cp, use_tc_tiling_on_sc=False)

    @functools.partial(
        pl.kernel,
        compiler_params=cp,
        out_type=jax.ShapeDtypeStruct((2, _NP, F), jnp.float32),
        mesh=mesh,
        scratch_types=[
            pltpu.VMEM((4, 128), jnp.int32),
            pltpu.VMEM((4, 128), jnp.int32),
            pltpu.VMEM((4, 128), jnp.float32),
            pltpu.VMEM((4, 128), jnp.float32),
            pltpu.VMEM((1, 128), jnp.int32),
            pltpu.VMEM((1, 128), jnp.int32),
            pltpu.VMEM((4 * _B, nw), jnp.int32),
            pltpu.VMEM((4 * _B, nw), jnp.int32),
            pltpu.VMEM((_B, F), jnp.float32),
            pltpu.VMEM((_B, F), jnp.float32),
            pltpu.VMEM_SHARED((_NP, F), jnp.float32),
            pltpu.SemaphoreType.DMA,
            pltpu.SemaphoreType.DMA,
            pltpu.SemaphoreType.DMA,
            pltpu.SemaphoreType.DMA,
            pltpu.SemaphoreType.DMA,
            pltpu.SemaphoreType.DMA,
            pltpu.SemaphoreType.DMA,
            pltpu.SemaphoreType.DMA,
            pltpu.SemaphoreType.DMA,
            pltpu.SemaphoreType.DMA,
        ],
    )
    def edge_kernel(h_hbm, gx_hbm, cf_hbm, dst_hbm, zero_hbm, out_hbm,
                    gx0, gx1, cf0, cf1, dx0, dx1, rw0, rw1, mg0, mg1, acc_sh,
                    s_gx0, s_gx1, s_cf0, s_cf1, s_dx0, s_dx1,
                    s_rw0, s_rw1, s_sc0, s_sc1):
        gxs = (gx0, gx1)
        cfs = (cf0, cf1)
        dxs = (dx0, dx1)
        rws = (rw0, rw1)
        mgs = (mg0, mg1)
        sgx = (s_gx0, s_gx1)
        scf = (s_cf0, s_cf1)
        sdx = (s_dx0, s_dx1)
        srw = (s_rw0, s_rw1)
        ssc = (s_sc0, s_sc1)
        cid = lax.axis_index("c")
        sid = lax.axis_index("s")
        stripe = sid * stripe_rows
        nch = jnp.where(cid == 0, k0, k1)
        base_chunk = jnp.where(cid == 0, sid * k0, 16 * k0 + sid * k1)
        last = base_chunk + nch - 1

        def gx_copy(b, r):
            return pltpu.make_async_copy(
                gx_hbm.at[pl.ds(r * 4, 4)], gxs[b], sgx[b])

        def cf_copy(b, r):
            return pltpu.make_async_copy(
                cf_hbm.at[pl.ds(r * 4, 4)], cfs[b], scf[b])

        def dx_copy(b, r):
            return pltpu.make_async_copy(
                dst_hbm.at[pl.ds(r, 1)], dxs[b], sdx[b])

        def row_copies(b):
            return [pltpu.make_async_copy(
                h_hbm.at[gxs[b].at[g]],
                rws[b].at[pl.ds(g * 128, 128)], srw[b]) for g in range(4)]

        def sc_copy(b):
            return pltpu.make_async_copy(mgs[b], acc_sh.at[dxs[b].at[0]],
                                         ssc[b])

        k_consts = [jnp.full((16,), k, jnp.int32) for k in range(4)]
        himask = jnp.full((16,), -65536, jnp.int32)

        def compute(b):
            rows_v = rws[b]
            msg_v = mgs[b]
            cf_v = cfs[b]

            @pl.loop(0, _B)
            def _edge(e):
                idx_e = jnp.full((16,), e, jnp.int32)
                cb = [plsc.load_gather(cf_v, [k_consts[k], idx_e])
                      for k in range(4)]
                base = 4 * e
                for g in range(ngrp):
                    lo = None
                    hi = None
                    for k in range(4):
                        w = rows_v[base + k, pl.ds(g * 16, 16)]
                        flo = plsc.bitcast(w << 16, jnp.float32)
                        fhi = plsc.bitcast(w & himask, jnp.float32)
                        lo = cb[k] * flo if lo is None else lo + cb[k] * flo
                        hi = cb[k] * fhi if hi is None else hi + cb[k] * fhi
                    msg_v[e, pl.ds(g * 32, 16)] = lo
                    msg_v[e, pl.ds(g * 32 + 16, 16)] = hi

        pltpu.sync_copy(zero_hbm.at[pl.ds(stripe, stripe_rows)],
                        acc_sh.at[pl.ds(stripe, stripe_rows)])
        plsc.subcore_barrier()

        gx_copy(0, base_chunk).start()
        cf_copy(0, base_chunk).start()
        cf_copy(1, base_chunk + 1).start()
        gx_copy(0, base_chunk).wait()
        for c in row_copies(0):
            c.start()
        gx_copy(1, base_chunk + 1).start()

        @pl.loop(0, nch // 2)
        def _pair(i):
            t = base_chunk + 2 * i

            def half(b, r, r_pre):
                for c in row_copies(b):
                    c.wait()
                gx_copy(b, r_pre).start()

                @pl.when(i > 0)
                def _():
                    sc_copy(b).wait()
                dx_copy(b, r).start()
                cf_copy(b, 0).wait()
                compute(b)
                dx_copy(b, r).wait()
                pltpu.async_copy(mgs[b], acc_sh.at[dxs[b].at[0]],
                                 ssc[b], add=True)
                cf_copy(b, r_pre).start()

            gx_copy(1, 0).wait()
            for c in row_copies(1):
                c.start()
            half(0, t, jnp.minimum(t + 2, last))
            gx_copy(0, 0).wait()
            for c in row_copies(0):
                c.start()
            half(1, t + 1, jnp.minimum(t + 3, last))

        for c in row_copies(0):
            c.wait()
        gx_copy(1, 0).wait()
        cf_copy(0, 0).wait()
        cf_copy(1, 0).wait()
        sc_copy(0).wait()
        sc_copy(1).wait()

        plsc.subcore_barrier()
        pltpu.sync_copy(acc_sh.at[pl.ds(stripe, stripe_rows)],
                        out_hbm.at[cid, pl.ds(stripe, stripe_rows)])

    return edge_kernel


_edge_call_64 = _make_edge_call(64, 94, 66)
_edge_call_32 = _make_edge_call(32, 106, 54)


def _pack_perm(fout, F):
    lo, hi = [], []
    for m in range(fout // 2):
        b, rem = divmod(m, F // 2)
        g, j = divmod(rem, 16)
        lo.append(b * F + 32 * g + j)
        hi.append(b * F + 32 * g + 16 + j)
    return lo + hi


def kernel(x, edge_index, edge_attr, W0, W1, W2, W3):
    src = edge_index[0]
    dst = edge_index[1]
    pad = _EPAD - _E
    srcp = jnp.pad(src, (0, pad)).reshape(_EROWS, 128)
    dstp = jnp.pad(dst, (0, pad), constant_values=_TRASH).reshape(_EROWS, 128)
    up = jnp.pad(edge_attr[:, 0], (0, pad), constant_values=-1.0)
    vp = jnp.pad(edge_attr[:, 1], (0, pad), constant_values=-1.0)

    gidx_k, coeff_k = _prep_call(srcp, up.reshape(_EROWS, 128),
                                 vp.reshape(_EROWS, 128))
    gidx_em = gidx_k.reshape(4, _EPAD).T.reshape(_EPAD * 4 // 128, 128)
    coeff_km = coeff_k.transpose(1, 0, 2).reshape(_EROWS * 4, 128)

    x_pad = jnp.pad(x, ((0, _NP - _N), (0, 0)))
    dims = [128, 64, 64, 64, 32]
    Ws = [W0, W1, W2, W3]
    w2ds = [
        Ws[i].reshape(_T * _T, dims[i], dims[i + 1])
        .transpose(1, 0, 2).reshape(dims[i], _T * _T * dims[i + 1])
        [:, jnp.array(_pack_perm(_T * _T * dims[i + 1], dims[i + 1]))]
        for i in range(4)
    ]
    z64 = jnp.zeros((_NP, 64), jnp.float32)
    z32 = jnp.zeros((_NP, 32), jnp.float32)

    P = None
    for i in range(4):
        fout = _T * _T * dims[i + 1]
        if i == 0:
            H = _make_mm_first(dims[0], fout)(x_pad, w2ds[0])
        else:
            scale = _SCALE if i == 3 else 1.0
            H = _make_mm_fused(dims[i], fout, scale)(P, w2ds[i])
        h_bits = H.reshape(_NP * 16, dims[i + 1] // 2)
        if dims[i + 1] == 64:
            P = _edge_call_64(h_bits, gidx_em, coeff_km, dstp, z64)
        else:
            P = _edge_call_32(h_bits, gidx_em, coeff_km, dstp, z32)

    out_full = _final_add(P)
    return out_full[:_N]


# speedup vs baseline: 1.0202x; 1.0202x over previous
import dataclasses
import functools

import jax
import jax.numpy as jnp
from jax import lax
from jax.experimental import pallas as pl
from jax.experimental.pallas import tpu as pltpu
from jax.experimental.pallas import tpu_sc as plsc

_N = 10000
_E = 320000
_T = 4
_NP = 10240
_TRASH = 10000
_NW = 32
_B = 128
_NCHUNK = 80
_EPW = _NCHUNK * _B
_EPAD = _NW * _EPW
_EROWS = _EPAD // 128
_SCALE = 1.0 / 128.0


def _prep_body(src_ref, u_ref, v_ref, gidx_ref, coeff_ref):
    inv_sp = (_T - 1) / 2.0
    tu = (u_ref[...] + 1.0) * inv_sp
    p = jnp.clip(jnp.floor(tu), 0.0, _T - 2.0)
    fu = tu - p
    tv = (v_ref[...] + 1.0) * inv_sp
    q = jnp.clip(jnp.floor(tv), 0.0, _T - 2.0)
    fv = tv - q
    base = src_ref[...] * 16 + p.astype(jnp.int32) * 4 + q.astype(jnp.int32)
    gu = 1.0 - fu
    gv = 1.0 - fv
    gidx_ref[0] = base
    gidx_ref[1] = base + 1
    gidx_ref[2] = base + 4
    gidx_ref[3] = base + 5
    coeff_ref[0] = gu * gv
    coeff_ref[1] = gu * fv
    coeff_ref[2] = fu * gv
    coeff_ref[3] = fu * fv


_prep_call = pl.pallas_call(
    _prep_body,
    out_shape=(
        jax.ShapeDtypeStruct((4, _EROWS, 128), jnp.int32),
        jax.ShapeDtypeStruct((4, _EROWS, 128), jnp.float32),
    ),
)


def _pack_words(o):
    half = o.shape[-1] // 2
    lo = o[:, :half].astype(jnp.bfloat16).astype(jnp.float32)
    hi = o[:, half:].astype(jnp.bfloat16).astype(jnp.float32)
    lo_b = lax.shift_right_logical(
        lax.bitcast_convert_type(lo, jnp.int32), 16)
    hi_b = lax.bitcast_convert_type(hi, jnp.int32) & jnp.int32(-65536)
    return lo_b | hi_b


def _mm_first_body(a_ref, w_ref, o_ref):
    o = jnp.dot(a_ref[...], w_ref[...], preferred_element_type=jnp.float32)
    o_ref[...] = _pack_words(o)


def _mm_fused_body(p_ref, w_ref, o_ref, *, scale):
    a = jnp.maximum(p_ref[0] + p_ref[1], 0.0)
    o = jnp.dot(a, w_ref[...], preferred_element_type=jnp.float32)
    if scale != 1.0:
        o = o * scale
    o_ref[...] = _pack_words(o)


_MM_R = 1024


def _make_mm_first(din, fout):
    return pl.pallas_call(
        _mm_first_body,
        grid=(_NP // _MM_R,),
        in_specs=[
            pl.BlockSpec((_MM_R, din), lambda i: (i, 0)),
            pl.BlockSpec((din, fout), lambda i: (0, 0)),
        ],
        out_specs=pl.BlockSpec((_MM_R, fout // 2), lambda i: (i, 0)),
        out_shape=jax.ShapeDtypeStruct((_NP, fout // 2), jnp.int32),
    )


def _make_mm_fused(din, fout, scale):
    return pl.pallas_call(
        functools.partial(_mm_fused_body, scale=scale),
        grid=(_NP // _MM_R,),
        in_specs=[
            pl.BlockSpec((2, _MM_R, din), lambda i: (0, i, 0)),
            pl.BlockSpec((din, fout), lambda i: (0, 0)),
        ],
        out_specs=pl.BlockSpec((_MM_R, fout // 2), lambda i: (i, 0)),
        out_shape=jax.ShapeDtypeStruct((_NP, fout // 2), jnp.int32),
    )


def _final_add_body(p_ref, o_ref):
    o_ref[...] = p_ref[0] + p_ref[1]


_final_add = pl.pallas_call(
    _final_add_body,
    out_shape=jax.ShapeDtypeStruct((_NP, 32), jnp.float32),
)


def _make_edge_call(F, k0, k1):
    nw = F // 2
    ngrp = F // 32
    stripe_rows = _NP // 16
    mesh = plsc.VectorSubcoreMesh(
        core_axis_name="c", subcore_axis_name="s",
        num_cores=2, num_subcores=16)
    cp = pltpu.CompilerParams()
    if "needs_layout_passes" in pltpu.CompilerParams.__dataclass_fields__:
        cp = dataclasses.replace(cp, needs_layout_passes=False)
    if "use_tc_tiling_on_sc" in pltpu.CompilerParams.__dataclass_fields__:
        cp = dataclasses.replace(cp, use_tc_tiling_on_sc=False)

    @functools.partial(
        pl.kernel,
        compiler_params=cp,
        out_type=jax.ShapeDtypeStruct((2, _NP, F), jnp.float32),
        mesh=mesh,
        scratch_types=[
            pltpu.VMEM((4, 128), jnp.int32),
            pltpu.VMEM((4, 128), jnp.int32),
            pltpu.VMEM((4, 128), jnp.float32),
            pltpu.VMEM((4, 128), jnp.float32),
            pltpu.VMEM((1, 128), jnp.int32),
            pltpu.VMEM((1, 128), jnp.int32),
            pltpu.VMEM((4 * _B, nw), jnp.int32),
            pltpu.VMEM((4 * _B, nw), jnp.int32),
            pltpu.VMEM((_B, F), jnp.float32),
            pltpu.VMEM((_B, F), jnp.float32),
            pltpu.VMEM_SHARED((_NP, F), jnp.float32),
            pltpu.SemaphoreType.DMA,
            pltpu.SemaphoreType.DMA,
            pltpu.SemaphoreType.DMA,
            pltpu.SemaphoreType.DMA,
            pltpu.SemaphoreType.DMA,
            pltpu.SemaphoreType.DMA,
            pltpu.SemaphoreType.DMA,
            pltpu.SemaphoreType.DMA,
            pltpu.SemaphoreType.DMA,
            pltpu.SemaphoreType.DMA,
        ],
    )
    def edge_kernel(h_hbm, gx_hbm, cf_hbm, dst_hbm, zero_hbm, out_hbm,
                    gx0, gx1, cf0, cf1, dx0, dx1, rw0, rw1, mg0, mg1, acc_sh,
                    s_gx0, s_gx1, s_cf0, s_cf1, s_dx0, s_dx1,
                    s_rw0, s_rw1, s_sc0, s_sc1):
        gxs = (gx0, gx1)
        cfs = (cf0, cf1)
        dxs = (dx0, dx1)
        rws = (rw0, rw1)
        mgs = (mg0, mg1)
        sgx = (s_gx0, s_gx1)
        scf = (s_cf0, s_cf1)
        sdx = (s_dx0, s_dx1)
        srw = (s_rw0, s_rw1)
        ssc = (s_sc0, s_sc1)
        cid = lax.axis_index("c")
        sid = lax.axis_index("s")
        stripe = sid * stripe_rows
        nch = jnp.where(cid == 0, k0, k1)
        base_chunk = jnp.where(cid == 0, sid * k0, 16 * k0 + sid * k1)
        last = base_chunk + nch - 1

        def gx_copy(b, r):
            return pltpu.make_async_copy(
                gx_hbm.at[pl.ds(r * 4, 4)], gxs[b], sgx[b])

        def cf_copy(b, r):
            return pltpu.make_async_copy(
                cf_hbm.at[pl.ds(r * 4, 4)], cfs[b], scf[b])

        def dx_copy(b, r):
            return pltpu.make_async_copy(
                dst_hbm.at[pl.ds(r, 1)], dxs[b], sdx[b])

        def row_copies(b):
            return [pltpu.make_async_copy(
                h_hbm.at[gxs[b].at[g]],
                rws[b].at[pl.ds(g * 128, 128)], srw[b]) for g in range(4)]

        def sc_copy(b):
            return pltpu.make_async_copy(mgs[b], acc_sh.at[dxs[b].at[0]],
                                         ssc[b])

        k_consts = [jnp.full((16,), k, jnp.int32) for k in range(4)]
        himask = jnp.full((16,), -65536, jnp.int32)

        def compute(b):
            rows_v = rws[b]
            msg_v = mgs[b]
            cf_v = cfs[b]

            @pl.loop(0, _B)
            def _edge(e):
                idx_e = jnp.full((16,), e, jnp.int32)
                cb = [plsc.load_gather(cf_v, [k_consts[k], idx_e])
                      for k in range(4)]
                base = 4 * e
                for g in range(ngrp):
                    lo = None
                    hi = None
                    for k in range(4):
                        w = rows_v[base + k, pl.ds(g * 16, 16)]
                        flo = plsc.bitcast(w << 16, jnp.float32)
                        fhi = plsc.bitcast(w & himask, jnp.float32)
                        lo = cb[k] * flo if lo is None else lo + cb[k] * flo
                        hi = cb[k] * fhi if hi is None else hi + cb[k] * fhi
                    msg_v[e, pl.ds(g * 32, 16)] = lo
                    msg_v[e, pl.ds(g * 32 + 16, 16)] = hi

        pltpu.sync_copy(zero_hbm.at[pl.ds(stripe, stripe_rows)],
                        acc_sh.at[pl.ds(stripe, stripe_rows)])
        plsc.subcore_barrier()

        gx_copy(0, base_chunk).start()
        cf_copy(0, base_chunk).start()
        cf_copy(1, base_chunk + 1).start()
        gx_copy(0, base_chunk).wait()
        for c in row_copies(0):
            c.start()
        gx_copy(1, base_chunk + 1).start()

        @pl.loop(0, nch // 2)
        def _pair(i):
            t = base_chunk + 2 * i

            def half(b, r, r_pre):
                for c in row_copies(b):
                    c.wait()
                gx_copy(b, r_pre).start()

                @pl.when(i > 0)
                def _():
                    sc_copy(b).wait()
                dx_copy(b, r).start()
                cf_copy(b, 0).wait()
                compute(b)
                dx_copy(b, r).wait()
                pltpu.async_copy(mgs[b], acc_sh.at[dxs[b].at[0]],
                                 ssc[b], add=True)
                cf_copy(b, r_pre).start()

            gx_copy(1, 0).wait()
            for c in row_copies(1):
                c.start()
            half(0, t, jnp.minimum(t + 2, last))
            gx_copy(0, 0).wait()
            for c in row_copies(0):
                c.start()
            half(1, t + 1, jnp.minimum(t + 3, last))

        for c in row_copies(0):
            c.wait()
        gx_copy(1, 0).wait()
        cf_copy(0, 0).wait()
        cf_copy(1, 0).wait()
        sc_copy(0).wait()
        sc_copy(1).wait()

        plsc.subcore_barrier()
        pltpu.sync_copy(acc_sh.at[pl.ds(stripe, stripe_rows)],
                        out_hbm.at[cid, pl.ds(stripe, stripe_rows)])

    return edge_kernel


_edge_call_64 = _make_edge_call(64, 90, 70)
_edge_call_32 = _make_edge_call(32, 102, 58)


def _pack_perm(fout, F):
    lo, hi = [], []
    for m in range(fout // 2):
        b, rem = divmod(m, F // 2)
        g, j = divmod(rem, 16)
        lo.append(b * F + 32 * g + j)
        hi.append(b * F + 32 * g + 16 + j)
    return lo + hi


def kernel(x, edge_index, edge_attr, W0, W1, W2, W3):
    src = edge_index[0]
    dst = edge_index[1]
    pad = _EPAD - _E
    srcp = jnp.pad(src, (0, pad)).reshape(_EROWS, 128)
    dstp = jnp.pad(dst, (0, pad), constant_values=_TRASH).reshape(_EROWS, 128)
    up = jnp.pad(edge_attr[:, 0], (0, pad), constant_values=-1.0)
    vp = jnp.pad(edge_attr[:, 1], (0, pad), constant_values=-1.0)

    gidx_k, coeff_k = _prep_call(srcp, up.reshape(_EROWS, 128),
                                 vp.reshape(_EROWS, 128))
    gidx_em = gidx_k.reshape(4, _EPAD).T.reshape(_EPAD * 4 // 128, 128)
    coeff_km = coeff_k.transpose(1, 0, 2).reshape(_EROWS * 4, 128)

    x_pad = jnp.pad(x, ((0, _NP - _N), (0, 0)))
    dims = [128, 64, 64, 64, 32]
    Ws = [W0, W1, W2, W3]
    w2ds = [
        Ws[i].reshape(_T * _T, dims[i], dims[i + 1])
        .transpose(1, 0, 2).reshape(dims[i], _T * _T * dims[i + 1])
        [:, jnp.array(_pack_perm(_T * _T * dims[i + 1], dims[i + 1]))]
        for i in range(4)
    ]
    z64 = jnp.zeros((_NP, 64), jnp.float32)
    z32 = jnp.zeros((_NP, 32), jnp.float32)

    P = None
    for i in range(4):
        fout = _T * _T * dims[i + 1]
        if i == 0:
            H = _make_mm_first(dims[0], fout)(x_pad, w2ds[0])
        else:
            scale = _SCALE if i == 3 else 1.0
            H = _make_mm_fused(dims[i], fout, scale)(P, w2ds[i])
        h_bits = H.reshape(_NP * 16, dims[i + 1] // 2)
        if dims[i + 1] == 64:
            P = _edge_call_64(h_bits, gidx_em, coeff_km, dstp, z64)
        else:
            P = _edge_call_32(h_bits, gidx_em, coeff_km, dstp, z32)

    out_full = _final_add(P)
    return out_full[:_N]
